# baseline (speedup 1.0000x reference)
import functools

import jax
import jax.numpy as jnp
from jax import lax
from jax.experimental import pallas as pl
from jax.experimental.pallas import tpu as pltpu
from jax.experimental.pallas import tpu_sc as plsc

_LANES = 16
_NBUF = 4


def _sc_sum(word_indices, emb_table):
    B, S = word_indices.shape
    _, E = emb_table.shape
    NC, NS = 2, 16
    NW = NC * NS
    R = B // NW
    EV = E // _LANES
    C0 = 104
    C1 = S - C0
    assert C0 % 8 == 0 and C1 <= 128 and S % 8 == 0
    NB = _NBUF
    MAIN = (R // NB) * NB

    mesh = plsc.VectorSubcoreMesh(core_axis_name="c", subcore_axis_name="s")

    @functools.partial(
        pl.kernel,
        out_type=jax.ShapeDtypeStruct((B, E), jnp.float32),
        mesh=mesh,
        compiler_params=pltpu.CompilerParams(
            use_tc_tiling_on_sc=False, needs_layout_passes=False),
        scratch_types=[
            pltpu.VMEM((R, S), jnp.int32),
            pltpu.VMEM((NB, S, E), jnp.float32),
            pltpu.VMEM((NB, E), jnp.float32),
        ] + [pltpu.SemaphoreType.DMA] * (2 * NB),
    )
    def k(idx_hbm, table_hbm, out_hbm, idx_v, rows_v, sums_v, *allsems):
        sems = allsems[:NB]
        osems = allsems[NB:]
        wid = lax.axis_index("s") * NC + lax.axis_index("c")
        base = wid * R

        pltpu.sync_copy(idx_hbm.at[pl.ds(base, R)], idx_v)

        def gather_row(row, buf):
            sem = sems[buf]
            pltpu.async_copy(
                table_hbm.at[idx_v.at[row, pl.ds(0, C0)]],
                rows_v.at[buf, pl.ds(0, C0), :], sem)
            pltpu.async_copy(
                table_hbm.at[idx_v.at[row, pl.ds(C0, C1)]],
                rows_v.at[buf, pl.ds(C0, C1), :], sem)

        def wait_row(buf):
            sem = sems[buf]
            pltpu.make_async_copy(
                table_hbm.at[idx_v.at[0, pl.ds(0, C0)]],
                rows_v.at[buf, pl.ds(0, C0), :], sem).wait()
            pltpu.make_async_copy(
                table_hbm.at[idx_v.at[0, pl.ds(C0, C1)]],
                rows_v.at[buf, pl.ds(C0, C1), :], sem).wait()

        def accumulate(row, buf):
            def body(t, acc):
                acc = list(acc)
                for j in range(8):
                    s = t * 8 + j
                    for e in range(EV):
                        acc[e] = acc[e] + rows_v[
                            buf, s, pl.ds(e * _LANES, _LANES)]
                return tuple(acc)
            acc = lax.fori_loop(
                0, S // 8, body,
                tuple(jnp.zeros((_LANES,), jnp.float32) for _ in range(EV)))
            for e in range(EV):
                sums_v[buf, pl.ds(e * _LANES, _LANES)] = acc[e]
            pltpu.async_copy(
                sums_v.at[buf], out_hbm.at[base + row], osems[buf])

        def wait_sums(buf):
            pltpu.make_async_copy(
                sums_v.at[buf], out_hbm.at[0], osems[buf]).wait()

        for b in range(NB):
            gather_row(b, b)

        def outer(g, carry):
            for b in range(NB):
                row = g + b
                wait_row(b)
                @pl.when(g > 0)
                def _ws(b=b):
                    wait_sums(b)
                accumulate(row, b)
                gather_row(jnp.minimum(row + NB, R - 1), b)
            return carry

        lax.fori_loop(0, MAIN // NB, lambda t, c: outer(t * NB, c), 0)

        for b in range(NB):
            row = MAIN + b
            wait_row(b)
            wait_sums(b)
            if row < R:
                accumulate(row, b)
                wait_sums(b)

    return k(word_indices, emb_table)


def _tc_finish(sums, word_indices, emb0, W1, b1, W2, b2):
    B, S = word_indices.shape
    E = sums.shape[1]
    H = W1.shape[1]
    C = W2.shape[1]
    BB = B
    grid = B // BB

    def body(sums_ref, idx_ref, emb0_ref, w1_ref, b1_ref, w2_ref, b2_ref,
             out_ref):
        idx = idx_ref[...]
        cnt = jnp.sum((idx != 0).astype(jnp.float32), axis=1, keepdims=True)
        npad = float(S) - cnt
        summed = sums_ref[...] - npad * emb0_ref[...]
        avg = jnp.where(cnt > 0, summed / jnp.maximum(cnt, 1.0), 0.0)
        hidden = jnp.maximum(
            jnp.dot(avg, w1_ref[...], preferred_element_type=jnp.float32)
            + b1_ref[...], 0.0)
        out_ref[...] = (
            jnp.dot(hidden, w2_ref[...], preferred_element_type=jnp.float32)
            + b2_ref[...])

    return pl.pallas_call(
        body,
        grid=(grid,),
        in_specs=[
            pl.BlockSpec((BB, E), lambda i: (i, 0)),
            pl.BlockSpec((BB, S), lambda i: (i, 0)),
            pl.BlockSpec((1, E), lambda i: (0, 0)),
            pl.BlockSpec((E, H), lambda i: (0, 0)),
            pl.BlockSpec((1, H), lambda i: (0, 0)),
            pl.BlockSpec((H, C), lambda i: (0, 0)),
            pl.BlockSpec((1, C), lambda i: (0, 0)),
        ],
        out_specs=pl.BlockSpec((BB, C), lambda i: (i, 0)),
        out_shape=jax.ShapeDtypeStruct((B, C), jnp.float32),
    )(sums, word_indices, emb0, W1, b1, W2, b2)


def kernel(word_indices, emb_table, W1, b1, W2, b2):
    idx = word_indices.astype(jnp.int32)
    sums = _sc_sum(idx, emb_table)
    return _tc_finish(sums, idx, emb_table[0:1], W1, b1.reshape(1, -1),
                      W2, b2.reshape(1, -1))


# speedup vs baseline: 1.0299x; 1.0299x over previous
import functools

import jax
import jax.numpy as jnp
from jax import lax
from jax.experimental import pallas as pl
from jax.experimental.pallas import tpu as pltpu
from jax.experimental.pallas import tpu_sc as plsc

_LANES = 16
_NBUF = 3


def _sc_sum(word_indices, emb_table):
    B, S = word_indices.shape
    _, E = emb_table.shape
    NC, NS = 2, 16
    NW = NC * NS
    R = B // NW
    EV = E // _LANES
    C0 = 104
    C1 = S - C0
    assert C0 % 8 == 0 and C1 <= 128 and S % 8 == 0
    NB = _NBUF
    MAIN = (R // NB) * NB

    mesh = plsc.VectorSubcoreMesh(core_axis_name="c", subcore_axis_name="s")

    @functools.partial(
        pl.kernel,
        out_type=jax.ShapeDtypeStruct((B, E), jnp.float32),
        mesh=mesh,
        compiler_params=pltpu.CompilerParams(
            use_tc_tiling_on_sc=False, needs_layout_passes=False),
        scratch_types=[
            pltpu.VMEM((R, S), jnp.int32),
            pltpu.VMEM((NB, S, E), jnp.float32),
            pltpu.VMEM((R, E), jnp.float32),
        ] + [pltpu.SemaphoreType.DMA] * NB,
    )
    def k(idx_hbm, table_hbm, out_hbm, idx_v, rows_v, sums_v, *sems):
        wid = lax.axis_index("s") * NC + lax.axis_index("c")
        base = wid * R

        pltpu.sync_copy(idx_hbm.at[pl.ds(base, R)], idx_v)

        def gather_row(row, buf):
            sem = sems[buf]
            pltpu.async_copy(
                table_hbm.at[idx_v.at[row, pl.ds(0, C0)]],
                rows_v.at[buf, pl.ds(0, C0), :], sem)
            pltpu.async_copy(
                table_hbm.at[idx_v.at[row, pl.ds(C0, C1)]],
                rows_v.at[buf, pl.ds(C0, C1), :], sem)

        def wait_row(buf):
            sem = sems[buf]
            pltpu.make_async_copy(
                table_hbm.at[idx_v.at[0, pl.ds(0, C0)]],
                rows_v.at[buf, pl.ds(0, C0), :], sem).wait()
            pltpu.make_async_copy(
                table_hbm.at[idx_v.at[0, pl.ds(C0, C1)]],
                rows_v.at[buf, pl.ds(C0, C1), :], sem).wait()

        def accumulate(row, buf):
            def body(t, acc):
                acc = list(acc)
                for j in range(8):
                    s = t * 8 + j
                    for e in range(EV):
                        acc[e] = acc[e] + rows_v[
                            buf, s, pl.ds(e * _LANES, _LANES)]
                return tuple(acc)
            acc = lax.fori_loop(
                0, S // 8, body,
                tuple(jnp.zeros((_LANES,), jnp.float32) for _ in range(EV)))
            for e in range(EV):
                sums_v[row, pl.ds(e * _LANES, _LANES)] = acc[e]

        for b in range(NB):
            gather_row(b, b)

        def outer(g, carry):
            for b in range(NB):
                row = g + b
                wait_row(b)
                accumulate(row, b)
                gather_row(jnp.minimum(row + NB, R - 1), b)
            return carry

        lax.fori_loop(0, MAIN // NB, lambda t, c: outer(t * NB, c), 0)

        for b in range(NB):
            row = MAIN + b
            wait_row(b)
            if row < R:
                accumulate(row, b)

        pltpu.sync_copy(sums_v, out_hbm.at[pl.ds(base, R)])

    return k(word_indices, emb_table)


def _tc_finish(sums, word_indices, emb0, W1, b1, W2, b2):
    B, S = word_indices.shape
    E = sums.shape[1]
    H = W1.shape[1]
    C = W2.shape[1]
    BB = B
    grid = B // BB

    def body(sums_ref, idx_ref, emb0_ref, w1_ref, b1_ref, w2_ref, b2_ref,
             out_ref):
        idx = idx_ref[...]
        cnt = jnp.sum((idx != 0).astype(jnp.float32), axis=1, keepdims=True)
        npad = float(S) - cnt
        summed = sums_ref[...] - npad * emb0_ref[...]
        avg = jnp.where(cnt > 0, summed / jnp.maximum(cnt, 1.0), 0.0)
        hidden = jnp.maximum(
            jnp.dot(avg, w1_ref[...], preferred_element_type=jnp.float32)
            + b1_ref[...], 0.0)
        out_ref[...] = (
            jnp.dot(hidden, w2_ref[...], preferred_element_type=jnp.float32)
            + b2_ref[...])

    return pl.pallas_call(
        body,
        grid=(grid,),
        in_specs=[
            pl.BlockSpec((BB, E), lambda i: (i, 0)),
            pl.BlockSpec((BB, S), lambda i: (i, 0)),
            pl.BlockSpec((1, E), lambda i: (0, 0)),
            pl.BlockSpec((E, H), lambda i: (0, 0)),
            pl.BlockSpec((1, H), lambda i: (0, 0)),
            pl.BlockSpec((H, C), lambda i: (0, 0)),
            pl.BlockSpec((1, C), lambda i: (0, 0)),
        ],
        out_specs=pl.BlockSpec((BB, C), lambda i: (i, 0)),
        out_shape=jax.ShapeDtypeStruct((B, C), jnp.float32),
    )(sums, word_indices, emb0, W1, b1, W2, b2)


def kernel(word_indices, emb_table, W1, b1, W2, b2):
    idx = word_indices.astype(jnp.int32)
    sums = _sc_sum(idx, emb_table)
    return _tc_finish(sums, idx, emb_table[0:1], W1, b1.reshape(1, -1),
                      W2, b2.reshape(1, -1))
